# baseline (speedup 1.0000x reference)
import jax
import jax.numpy as jnp
from jax.experimental import pallas as pl
from jax.experimental.pallas import tpu as pltpu

_EPS = 1e-8


def _simrel_kernel(x_hbm, ca_ref, out_ref, x_vmem, sem):
    nb = x_hbm.shape[0]
    for i in range(nb):
        pltpu.make_async_copy(x_hbm.at[i], x_vmem.at[i], sem.at[i]).start()
    ca = ca_ref[...]
    inv_ca = 1.0 / jnp.maximum(jnp.sqrt(jnp.sum(ca * ca, axis=1, keepdims=True)), _EPS)
    for i in range(nb):
        pltpu.make_async_copy(x_hbm.at[i], x_vmem.at[i], sem.at[i]).wait()
        x = x_vmem[i]
        inv_in = 1.0 / jnp.maximum(jnp.sqrt(jnp.sum(x * x, axis=1)), _EPS)
        dots = jax.lax.dot_general(
            ca, x,
            dimension_numbers=(((1,), (1,)), ((), ())),
            preferred_element_type=jnp.float32,
        )
        out_ref[i] = dots * inv_ca * inv_in[None, :]


def kernel(inputs, labels, class_avgs):
    del labels
    b, s, d = inputs.shape
    k = class_avgs.shape[0]
    out_t = pl.pallas_call(
        _simrel_kernel,
        in_specs=[
            pl.BlockSpec(memory_space=pltpu.MemorySpace.HBM),
            pl.BlockSpec(memory_space=pltpu.MemorySpace.VMEM),
        ],
        out_specs=pl.BlockSpec(memory_space=pltpu.MemorySpace.VMEM),
        out_shape=jax.ShapeDtypeStruct((b, k, s), jnp.float32),
        scratch_shapes=[
            pltpu.VMEM((b, s, d), jnp.float32),
            pltpu.SemaphoreType.DMA((b,)),
        ],
    )(inputs, class_avgs)
    return jnp.swapaxes(out_t, 1, 2)


# speedup vs baseline: 1.3840x; 1.3840x over previous
import jax
import jax.numpy as jnp
from jax.experimental import pallas as pl
from jax.experimental.pallas import tpu as pltpu

_EPS = 1e-8


def _simrel_kernel(x_ref, ca_ref, out_ref):
    b = x_ref.shape[0]
    ca = ca_ref[...]
    inv_ca = 1.0 / jnp.maximum(jnp.sqrt(jnp.sum(ca * ca, axis=1, keepdims=True)), _EPS)
    for i in range(b):
        x = x_ref[i]
        inv_in = 1.0 / jnp.maximum(jnp.sqrt(jnp.sum(x * x, axis=1)), _EPS)
        dots = jax.lax.dot_general(
            ca, x,
            dimension_numbers=(((1,), (1,)), ((), ())),
            preferred_element_type=jnp.float32,
        )
        out_ref[i] = dots * inv_ca * inv_in[None, :]


def kernel(inputs, labels, class_avgs):
    del labels
    b, s, d = inputs.shape
    k = class_avgs.shape[0]
    out_t = pl.pallas_call(
        _simrel_kernel,
        out_shape=jax.ShapeDtypeStruct((b, k, s), jnp.float32),
        compiler_params=pltpu.CompilerParams(skip_device_barrier=True),
    )(inputs, class_avgs)
    return jnp.swapaxes(out_t, 1, 2)


# speedup vs baseline: 1.3905x; 1.0047x over previous
import jax
import jax.numpy as jnp
from jax.experimental import pallas as pl
from jax.experimental.pallas import tpu as pltpu

_EPS = 1e-8


def _simrel_kernel(x_ref, ca_ref, out_hbm, o_vmem, sem):
    b = x_ref.shape[0]
    ca = ca_ref[...]
    inv_ca = 1.0 / jnp.maximum(jnp.sqrt(jnp.sum(ca * ca, axis=1, keepdims=True)), _EPS)
    for i in range(b):
        x = x_ref[i]
        inv_in = 1.0 / jnp.maximum(jnp.sqrt(jnp.sum(x * x, axis=1)), _EPS)
        dots = jax.lax.dot_general(
            ca, x,
            dimension_numbers=(((1,), (1,)), ((), ())),
            preferred_element_type=jnp.float32,
        )
        o_vmem[i] = dots * inv_ca * inv_in[None, :]
        pltpu.make_async_copy(o_vmem.at[i], out_hbm.at[i], sem.at[i]).start()
    for i in range(b):
        pltpu.make_async_copy(o_vmem.at[i], out_hbm.at[i], sem.at[i]).wait()


def kernel(inputs, labels, class_avgs):
    del labels
    b, s, d = inputs.shape
    k = class_avgs.shape[0]
    out_t = pl.pallas_call(
        _simrel_kernel,
        in_specs=[
            pl.BlockSpec(memory_space=pltpu.MemorySpace.VMEM),
            pl.BlockSpec(memory_space=pltpu.MemorySpace.VMEM),
        ],
        out_specs=pl.BlockSpec(memory_space=pltpu.MemorySpace.HBM),
        out_shape=jax.ShapeDtypeStruct((b, k, s), jnp.float32),
        scratch_shapes=[
            pltpu.VMEM((b, k, s), jnp.float32),
            pltpu.SemaphoreType.DMA((b,)),
        ],
    )(inputs, class_avgs)
    return jnp.swapaxes(out_t, 1, 2)
